# Initial kernel scaffold; baseline (speedup 1.0000x reference)
import jax
import jax.numpy as jnp
from jax.experimental import pallas as pl


def kernel(x, edge_index, W1, as1, ad1, b1, W2, as2, ad2, b2):
    raise NotImplementedError("write your pallas kernel here")


# speedup vs baseline: 47.9544x; 47.9544x over previous
import functools

import jax
import jax.numpy as jnp
from jax import lax
from jax.experimental import pallas as pl
from jax.experimental.pallas import tpu as pltpu
from jax.experimental.pallas import tpu_sc as plsc

NN = 10000
NPAD = 10240
EDGES = 320000
ETOT = EDGES + NN
NCORE = 2
NSUB = 16
NTILE = NCORE * NSUB
CHUNK = 128
CPT = -(-ETOT // (NTILE * CHUNK))
EPT = CPT * CHUNK
EPAD = EPT * NTILE
ROWS_PER_TILE = NPAD // NSUB

U1W = 80
H2W = 48
U2W = 48
BLK = 1024


def _tc_pre_body(x_ref, w1_ref, acat_ref, h_ref, asd_ref):
    h = jnp.dot(x_ref[...], w1_ref[...], preferred_element_type=jnp.float32)
    h_ref[...] = h
    asd_ref[...] = jnp.dot(h, acat_ref[...], preferred_element_type=jnp.float32)


def _tc_mid_body(u0_ref, u1_ref, b1_ref, w2p_ref, a2_ref, e16_ref,
                 h2_ref, asd2_ref):
    u = u0_ref[...] + u1_ref[...]
    den = jnp.dot(u[:, 64:80], e16_ref[...], preferred_element_type=jnp.float32)
    h1 = jnp.maximum(u[:, :64] / (den + 1e-16) + b1_ref[...], 0.0)
    h2 = jnp.dot(h1, w2p_ref[...], preferred_element_type=jnp.float32)
    h2_ref[...] = h2
    asd2_ref[...] = jnp.dot(h2, a2_ref[...], preferred_element_type=jnp.float32)


def _tc_post_body(u0_ref, u1_ref, b2_ref, e1_ref, out_ref):
    u = u0_ref[...] + u1_ref[...]
    den = jnp.dot(u[:, 40:48], e1_ref[...], preferred_element_type=jnp.float32)
    logits = u[:, :40] / (den + 1e-16) + b2_ref[...]
    m = jnp.max(logits, axis=1, keepdims=True)
    p = logits - m
    out_ref[...] = p - jnp.log(jnp.sum(jnp.exp(p), axis=1, keepdims=True))


def _row_spec(width):
    return pl.BlockSpec((BLK, width), lambda i: (i, 0))


def _full_spec(shape):
    return pl.BlockSpec(shape, lambda i: tuple(0 for _ in shape))


def _tc_pre(xp, w1, acat):
    return pl.pallas_call(
        _tc_pre_body,
        grid=(NPAD // BLK,),
        in_specs=[_row_spec(128), _full_spec((128, 64)), _full_spec((64, 16))],
        out_specs=[_row_spec(64), _row_spec(16)],
        out_shape=[jax.ShapeDtypeStruct((NPAD, 64), jnp.float32),
                   jax.ShapeDtypeStruct((NPAD, 16), jnp.float32)],
    )(xp, w1, acat)


def _tc_mid(u0, u1, b1r, w2p, a2, e16):
    return pl.pallas_call(
        _tc_mid_body,
        grid=(NPAD // BLK,),
        in_specs=[_row_spec(U1W), _row_spec(U1W), _full_spec((1, 64)),
                  _full_spec((64, H2W)), _full_spec((H2W, 16)),
                  _full_spec((16, 64))],
        out_specs=[_row_spec(H2W), _row_spec(16)],
        out_shape=[jax.ShapeDtypeStruct((NPAD, H2W), jnp.float32),
                   jax.ShapeDtypeStruct((NPAD, 16), jnp.float32)],
    )(u0, u1, b1r, w2p, a2, e16)


def _tc_post(u0, u1, b2r, e1):
    return pl.pallas_call(
        _tc_post_body,
        grid=(NPAD // BLK,),
        in_specs=[_row_spec(U2W), _row_spec(U2W), _full_spec((1, 40)),
                  _full_spec((8, 40))],
        out_specs=_row_spec(40),
        out_shape=jax.ShapeDtypeStruct((NPAD, 40), jnp.float32),
    )(u0, u1, b2r, e1)


def _sc_edge_kernel(width_feat, width_acc, per_edge_fn):
    def body(src_hbm, dst_hbm, asd_hbm, feat_hbm, out_hbm,
             src_v, dst_v, asds_v, asdd_v, frows_v, msg_v, acc_sh,
             sem_a, sem_b, sem_h):
        c = lax.axis_index("c")
        s = lax.axis_index("s")
        wid = c * NSUB + s

        pltpu.sync_copy(src_hbm.at[pl.ds(wid * CPT, CPT)], src_v)
        pltpu.sync_copy(dst_hbm.at[pl.ds(wid * CPT, CPT)], dst_v)

        def _zero_row(r, _):
            for k in range(width_acc // 16):
                msg_v[r, pl.ds(16 * k, 16)] = jnp.zeros((16,), jnp.float32)
            return 0
        lax.fori_loop(0, CHUNK, _zero_row, 0)
        for i in range(ROWS_PER_TILE // CHUNK):
            pltpu.sync_copy(msg_v,
                            acc_sh.at[pl.ds(s * ROWS_PER_TILE + i * CHUNK,
                                            CHUNK)])
        plsc.subcore_barrier()

        def chunk_body(j, _):
            ca = pltpu.async_copy(asd_hbm.at[src_v.at[j]], asds_v, sem_a)
            cb = pltpu.async_copy(asd_hbm.at[dst_v.at[j]], asdd_v, sem_b)
            ch = pltpu.async_copy(feat_hbm.at[src_v.at[j]], frows_v, sem_h)
            ca.wait()
            cb.wait()
            ch.wait()

            def edge_body(e, _):
                per_edge_fn(e, asds_v, asdd_v, frows_v, msg_v)
                return 0
            lax.fori_loop(0, CHUNK, edge_body, 0)

            pltpu.sync_copy(msg_v, acc_sh.at[dst_v.at[j]], add=True)
            return 0

        lax.fori_loop(0, CPT, chunk_body, 0)
        plsc.subcore_barrier()

        pltpu.sync_copy(acc_sh.at[pl.ds(s * ROWS_PER_TILE, ROWS_PER_TILE)],
                        out_hbm.at[c, pl.ds(s * ROWS_PER_TILE, ROWS_PER_TILE)])

    mesh = plsc.VectorSubcoreMesh(core_axis_name="c", subcore_axis_name="s",
                                  num_cores=NCORE, num_subcores=NSUB)
    return pl.kernel(
        body,
        out_type=jax.ShapeDtypeStruct((NCORE, NPAD, width_acc), jnp.float32),
        mesh=mesh,
        compiler_params=pltpu.CompilerParams(use_tc_tiling_on_sc=False),
        scratch_types=[
            pltpu.VMEM((CPT, CHUNK), jnp.int32),
            pltpu.VMEM((CPT, CHUNK), jnp.int32),
            pltpu.VMEM((CHUNK, 16), jnp.float32),
            pltpu.VMEM((CHUNK, 16), jnp.float32),
            pltpu.VMEM((CHUNK, width_feat), jnp.float32),
            pltpu.VMEM((CHUNK, width_acc), jnp.float32),
            pltpu.VMEM_SHARED((NPAD, width_acc), jnp.float32),
            pltpu.SemaphoreType.DMA,
            pltpu.SemaphoreType.DMA,
            pltpu.SemaphoreType.DMA,
        ],
    )


def _vgather(v, idx):
    return v.at[idx].get(mode="promise_in_bounds")


def _edge1_compute(e, asds_v, asdd_v, frows_v, msg_v):
    iota = lax.iota(jnp.int32, 16)
    idx_hi = (iota & 7) + 8
    srow = asds_v[e]
    drow = asdd_v[e]
    ev = srow + _vgather(drow, idx_hi)
    ev = jnp.where(ev > 0, ev, 0.2 * ev)
    w = jnp.where(iota < 8, jnp.exp(ev), 0.0)
    for k in range(4):
        hk = frows_v[e, pl.ds(16 * k, 16)]
        wk = _vgather(w, jnp.right_shift(iota, 3) + 2 * k)
        msg_v[e, pl.ds(16 * k, 16)] = hk * wk
    msg_v[e, pl.ds(64, 16)] = w


def _edge2_compute(e, asds_v, asdd_v, frows_v, msg_v):
    iota = lax.iota(jnp.int32, 16)
    z16 = iota * 0
    srow = asds_v[e]
    drow = asdd_v[e]
    ev = _vgather(srow, z16) + _vgather(drow, z16 + 1)
    ev = jnp.where(ev > 0, ev, 0.2 * ev)
    w = jnp.exp(ev)
    msg_v[e, pl.ds(0, 16)] = frows_v[e, pl.ds(0, 16)] * w
    msg_v[e, pl.ds(16, 16)] = frows_v[e, pl.ds(16, 16)] * w
    msg_v[e, pl.ds(32, 16)] = (frows_v[e, pl.ds(32, 16)] * w
                               + jnp.where(iota == 8, w, 0.0))


def kernel(x, edge_index, W1, as1, ad1, b1, W2, as2, ad2, b2):
    f32 = jnp.float32
    xp = jnp.zeros((NPAD, 128), f32).at[:NN].set(x)

    loop = jnp.arange(NN, dtype=jnp.int32)
    pad = jnp.full((EPAD - ETOT,), NN, jnp.int32)
    src = jnp.concatenate([edge_index[0], loop, pad]).reshape(NTILE * CPT,
                                                              CHUNK)
    dst = jnp.concatenate([edge_index[1], loop, pad]).reshape(NTILE * CPT,
                                                              CHUNK)

    j = jnp.arange(64)
    hd = j // 8
    acat = jnp.zeros((64, 16), f32)
    acat = acat.at[j, hd].set(as1.reshape(-1))
    acat = acat.at[j, hd + 8].set(ad1.reshape(-1))

    w2p = jnp.zeros((64, H2W), f32).at[:, :40].set(W2)
    a2 = jnp.zeros((H2W, 16), f32)
    a2 = a2.at[:40, 0].set(as2[0])
    a2 = a2.at[:40, 1].set(ad2[0])

    e16 = (jnp.arange(64)[None, :] // 8
           == jnp.arange(16)[:, None]).astype(f32)
    e1 = (jnp.arange(8)[:, None] == 0).astype(f32) * jnp.ones((8, 40), f32)

    h, asd = _tc_pre(xp, W1, acat)

    u1 = _sc_edge_kernel(64, U1W, _edge1_compute)(src, dst, asd, h)
    h2, asd2 = _tc_mid(u1[0], u1[1], b1.reshape(1, 64), w2p, a2, e16)

    u2 = _sc_edge_kernel(H2W, U2W, _edge2_compute)(src, dst, asd2, h2)
    out = _tc_post(u2[0], u2[1], b2.reshape(1, 40), e1)
    return out[:NN]


# speedup vs baseline: 90.4402x; 1.8860x over previous
import functools

import jax
import jax.numpy as jnp
from jax import lax
from jax.experimental import pallas as pl
from jax.experimental.pallas import tpu as pltpu
from jax.experimental.pallas import tpu_sc as plsc

NN = 10000
NPAD = 10240
EDGES = 320000
ETOT = EDGES + NN
NCORE = 2
NSUB = 16
NTILE = NCORE * NSUB
CHUNK = 128
CPT = -(-ETOT // (NTILE * CHUNK))
EPT = CPT * CHUNK
EPAD = EPT * NTILE
ROWS_PER_TILE = NPAD // NSUB

U1W = 80
H2W = 48
U2W = 48
BLK = 1024


def _tc_pre_body(x_ref, w1_ref, acat_ref, h_ref, asd_ref):
    h = jnp.dot(x_ref[...], w1_ref[...], preferred_element_type=jnp.float32)
    h_ref[...] = h
    asd_ref[...] = jnp.dot(h, acat_ref[...], preferred_element_type=jnp.float32)


def _tc_mid_body(u0_ref, u1_ref, b1_ref, w2p_ref, a2_ref, e16_ref,
                 h2_ref, asd2_ref):
    u = u0_ref[...] + u1_ref[...]
    den = jnp.dot(u[:, 64:80], e16_ref[...], preferred_element_type=jnp.float32)
    h1 = jnp.maximum(u[:, :64] / (den + 1e-16) + b1_ref[...], 0.0)
    h2 = jnp.dot(h1, w2p_ref[...], preferred_element_type=jnp.float32)
    h2_ref[...] = h2
    asd2_ref[...] = jnp.dot(h2, a2_ref[...], preferred_element_type=jnp.float32)


def _tc_post_body(u0_ref, u1_ref, b2_ref, e1_ref, out_ref):
    u = u0_ref[...] + u1_ref[...]
    den = jnp.dot(u[:, 40:48], e1_ref[...], preferred_element_type=jnp.float32)
    logits = u[:, :40] / (den + 1e-16) + b2_ref[...]
    m = jnp.max(logits, axis=1, keepdims=True)
    p = logits - m
    out_ref[...] = p - jnp.log(jnp.sum(jnp.exp(p), axis=1, keepdims=True))


def _row_spec(width):
    return pl.BlockSpec((BLK, width), lambda i: (i, 0))


def _full_spec(shape):
    return pl.BlockSpec(shape, lambda i: tuple(0 for _ in shape))


def _tc_pre(xp, w1, acat):
    return pl.pallas_call(
        _tc_pre_body,
        grid=(NPAD // BLK,),
        in_specs=[_row_spec(128), _full_spec((128, 64)), _full_spec((64, 16))],
        out_specs=[_row_spec(64), _row_spec(16)],
        out_shape=[jax.ShapeDtypeStruct((NPAD, 64), jnp.float32),
                   jax.ShapeDtypeStruct((NPAD, 16), jnp.float32)],
    )(xp, w1, acat)


def _tc_mid(u0, u1, b1r, w2p, a2, e16):
    return pl.pallas_call(
        _tc_mid_body,
        grid=(NPAD // BLK,),
        in_specs=[_row_spec(U1W), _row_spec(U1W), _full_spec((1, 64)),
                  _full_spec((64, H2W)), _full_spec((H2W, 16)),
                  _full_spec((16, 64))],
        out_specs=[_row_spec(H2W), _row_spec(16)],
        out_shape=[jax.ShapeDtypeStruct((NPAD, H2W), jnp.float32),
                   jax.ShapeDtypeStruct((NPAD, 16), jnp.float32)],
    )(u0, u1, b1r, w2p, a2, e16)


def _tc_post(u0, u1, b2r, e1):
    return pl.pallas_call(
        _tc_post_body,
        grid=(NPAD // BLK,),
        in_specs=[_row_spec(U2W), _row_spec(U2W), _full_spec((1, 40)),
                  _full_spec((8, 40))],
        out_specs=_row_spec(40),
        out_shape=jax.ShapeDtypeStruct((NPAD, 40), jnp.float32),
    )(u0, u1, b2r, e1)


def _sc_edge_kernel(width_feat, width_acc, per_edge_fn):
    def body(src_hbm, dst_hbm, asd_hbm, feat_hbm, out_hbm,
             src_v, dst_v, asds_v, asdd_v, frows_v, msg_v, acc_sh,
             sem_a, sem_b, sem_h):
        c = lax.axis_index("c")
        s = lax.axis_index("s")
        wid = c * NSUB + s

        pltpu.sync_copy(src_hbm.at[pl.ds(wid * CPT, CPT)], src_v)
        pltpu.sync_copy(dst_hbm.at[pl.ds(wid * CPT, CPT)], dst_v)

        @plsc.parallel_loop(0, CHUNK, 1, unroll=4)
        def _zero_row(r):
            for k in range(width_acc // 16):
                msg_v[r, pl.ds(16 * k, 16)] = jnp.zeros((16,), jnp.float32)
        for i in range(ROWS_PER_TILE // CHUNK):
            pltpu.sync_copy(msg_v,
                            acc_sh.at[pl.ds(s * ROWS_PER_TILE + i * CHUNK,
                                            CHUNK)])
        plsc.subcore_barrier()

        def chunk_body(j, _):
            ca = pltpu.async_copy(asd_hbm.at[src_v.at[j]], asds_v, sem_a)
            cb = pltpu.async_copy(asd_hbm.at[dst_v.at[j]], asdd_v, sem_b)
            ch = pltpu.async_copy(feat_hbm.at[src_v.at[j]], frows_v, sem_h)
            ca.wait()
            cb.wait()
            ch.wait()

            @plsc.parallel_loop(0, CHUNK, 1, unroll=4)
            def edge_body(e):
                per_edge_fn(e, asds_v, asdd_v, frows_v, msg_v)

            pltpu.sync_copy(msg_v, acc_sh.at[dst_v.at[j]], add=True)
            return 0

        lax.fori_loop(0, CPT, chunk_body, 0)
        plsc.subcore_barrier()

        pltpu.sync_copy(acc_sh.at[pl.ds(s * ROWS_PER_TILE, ROWS_PER_TILE)],
                        out_hbm.at[c, pl.ds(s * ROWS_PER_TILE, ROWS_PER_TILE)])

    mesh = plsc.VectorSubcoreMesh(core_axis_name="c", subcore_axis_name="s",
                                  num_cores=NCORE, num_subcores=NSUB)
    return pl.kernel(
        body,
        out_type=jax.ShapeDtypeStruct((NCORE, NPAD, width_acc), jnp.float32),
        mesh=mesh,
        compiler_params=pltpu.CompilerParams(use_tc_tiling_on_sc=False),
        scratch_types=[
            pltpu.VMEM((CPT, CHUNK), jnp.int32),
            pltpu.VMEM((CPT, CHUNK), jnp.int32),
            pltpu.VMEM((CHUNK, 16), jnp.float32),
            pltpu.VMEM((CHUNK, 16), jnp.float32),
            pltpu.VMEM((CHUNK, width_feat), jnp.float32),
            pltpu.VMEM((CHUNK, width_acc), jnp.float32),
            pltpu.VMEM_SHARED((NPAD, width_acc), jnp.float32),
            pltpu.SemaphoreType.DMA,
            pltpu.SemaphoreType.DMA,
            pltpu.SemaphoreType.DMA,
        ],
    )


def _vgather(v, idx):
    return v.at[idx].get(mode="promise_in_bounds")


def _edge1_compute(e, asds_v, asdd_v, frows_v, msg_v):
    iota = lax.iota(jnp.int32, 16)
    idx_hi = (iota & 7) + 8
    srow = asds_v[e]
    drow = asdd_v[e]
    ev = srow + _vgather(drow, idx_hi)
    ev = jnp.where(ev > 0, ev, 0.2 * ev)
    w = jnp.where(iota < 8, jnp.exp(ev), 0.0)
    for k in range(4):
        hk = frows_v[e, pl.ds(16 * k, 16)]
        wk = _vgather(w, jnp.right_shift(iota, 3) + 2 * k)
        msg_v[e, pl.ds(16 * k, 16)] = hk * wk
    msg_v[e, pl.ds(64, 16)] = w


def _edge2_compute(e, asds_v, asdd_v, frows_v, msg_v):
    iota = lax.iota(jnp.int32, 16)
    z16 = iota * 0
    srow = asds_v[e]
    drow = asdd_v[e]
    ev = _vgather(srow, z16) + _vgather(drow, z16 + 1)
    ev = jnp.where(ev > 0, ev, 0.2 * ev)
    w = jnp.exp(ev)
    msg_v[e, pl.ds(0, 16)] = frows_v[e, pl.ds(0, 16)] * w
    msg_v[e, pl.ds(16, 16)] = frows_v[e, pl.ds(16, 16)] * w
    msg_v[e, pl.ds(32, 16)] = (frows_v[e, pl.ds(32, 16)] * w
                               + jnp.where(iota == 8, w, 0.0))


def kernel(x, edge_index, W1, as1, ad1, b1, W2, as2, ad2, b2):
    f32 = jnp.float32
    xp = jnp.zeros((NPAD, 128), f32).at[:NN].set(x)

    loop = jnp.arange(NN, dtype=jnp.int32)
    pad = jnp.full((EPAD - ETOT,), NN, jnp.int32)
    src = jnp.concatenate([edge_index[0], loop, pad]).reshape(NTILE * CPT,
                                                              CHUNK)
    dst = jnp.concatenate([edge_index[1], loop, pad]).reshape(NTILE * CPT,
                                                              CHUNK)

    j = jnp.arange(64)
    hd = j // 8
    acat = jnp.zeros((64, 16), f32)
    acat = acat.at[j, hd].set(as1.reshape(-1))
    acat = acat.at[j, hd + 8].set(ad1.reshape(-1))

    w2p = jnp.zeros((64, H2W), f32).at[:, :40].set(W2)
    a2 = jnp.zeros((H2W, 16), f32)
    a2 = a2.at[:40, 0].set(as2[0])
    a2 = a2.at[:40, 1].set(ad2[0])

    e16 = (jnp.arange(64)[None, :] // 8
           == jnp.arange(16)[:, None]).astype(f32)
    e1 = (jnp.arange(8)[:, None] == 0).astype(f32) * jnp.ones((8, 40), f32)

    h, asd = _tc_pre(xp, W1, acat)

    u1 = _sc_edge_kernel(64, U1W, _edge1_compute)(src, dst, asd, h)
    h2, asd2 = _tc_mid(u1[0], u1[1], b1.reshape(1, 64), w2p, a2, e16)

    u2 = _sc_edge_kernel(H2W, U2W, _edge2_compute)(src, dst, asd2, h2)
    out = _tc_post(u2[0], u2[1], b2.reshape(1, 40), e1)
    return out[:NN]
